# Initial kernel scaffold; baseline (speedup 1.0000x reference)
import jax
import jax.numpy as jnp
from jax.experimental import pallas as pl


def kernel(atomic_numbers, pos, edge_index, params):
    raise NotImplementedError("write your pallas kernel here")


# speedup vs baseline: 2.9326x; 2.9326x over previous
import functools
import math

import jax
import jax.numpy as jnp
from jax import lax
from jax.experimental import pallas as pl
from jax.experimental.pallas import tpu as pltpu
from jax.experimental.pallas import tpu_sc as plsc

N = 50000
E = 800000
H = 64
HH = H // 2
NB = 8
NUM_ATOMS = 100
CUTOFF = 5.0

NC = 2
NS = 16
NW = NC * NS
CH = 128
NCHUNKS = E // CH
ROWS_A = 3128
ROWS_LAST = N - (NS - 1) * ROWS_A

BN = 2000
BR = 64
EP = 819200
ER = EP // 128

_SC_MESH = plsc.VectorSubcoreMesh(
    core_axis_name="c", subcore_axis_name="s", num_cores=NC, num_subcores=NS
)


def _silu(x):
    return x * (1.0 / (1.0 + jnp.exp(-x)))


def _sc_edge_d2_body(row_h, col_h, px_h, py_h, pz_h, d2_h,
                     rowi, coli, rx, ry, rz, cx, cy, cz, d2v):
    c = lax.axis_index("c")
    s = lax.axis_index("s")
    wid = s * NC + c
    nmy = (NCHUNKS // NW) + (wid < (NCHUNKS % NW)).astype(jnp.int32)

    @pl.loop(0, nmy)
    def _chunk(i):
        chunk = wid + i * NW
        base = pl.multiple_of(chunk * CH, CH)
        pltpu.sync_copy(row_h.at[pl.ds(base, CH)], rowi)
        pltpu.sync_copy(col_h.at[pl.ds(base, CH)], coli)
        pltpu.sync_copy(px_h.at[rowi], rx)
        pltpu.sync_copy(py_h.at[rowi], ry)
        pltpu.sync_copy(pz_h.at[rowi], rz)
        pltpu.sync_copy(px_h.at[coli], cx)
        pltpu.sync_copy(py_h.at[coli], cy)
        pltpu.sync_copy(pz_h.at[coli], cz)
        for j in range(CH // 16):
            sl = pl.ds(j * 16, 16)
            dx = rx[sl] - cx[sl]
            dy = ry[sl] - cy[sl]
            dz = rz[sl] - cz[sl]
            d2v[sl] = dx * dx + dy * dy + dz * dz
        pltpu.sync_copy(d2v, d2_h.at[pl.ds(base, CH)])


_SC_PARAMS = pltpu.CompilerParams(use_tc_tiling_on_sc=False)

_sc_edge_d2 = functools.partial(
    pl.kernel,
    out_type=jax.ShapeDtypeStruct((EP,), jnp.float32),
    mesh=_SC_MESH,
    compiler_params=_SC_PARAMS,
    scratch_types=[
        pltpu.VMEM((CH,), jnp.int32),
        pltpu.VMEM((CH,), jnp.int32),
        pltpu.VMEM((CH,), jnp.float32),
        pltpu.VMEM((CH,), jnp.float32),
        pltpu.VMEM((CH,), jnp.float32),
        pltpu.VMEM((CH,), jnp.float32),
        pltpu.VMEM((CH,), jnp.float32),
        pltpu.VMEM((CH,), jnp.float32),
        pltpu.VMEM((CH,), jnp.float32),
    ],
)(_sc_edge_d2_body)


def _sc_messages_body(row_h, col_h, nf_h, rw_h, zer_h, agg_h,
                      rowi, coli, nj, rwv, agg_sh):
    c = lax.axis_index("c")
    s = lax.axis_index("s")
    rbase = pl.multiple_of(s * ROWS_A, 8)

    @pl.when(s < NS - 1)
    def _():
        pltpu.sync_copy(zer_h, agg_sh.at[pl.ds(rbase, ROWS_A)])

    @pl.when(s == NS - 1)
    def _():
        pltpu.sync_copy(zer_h.at[pl.ds(0, ROWS_LAST)],
                        agg_sh.at[pl.ds(rbase, ROWS_LAST)])

    plsc.subcore_barrier()

    nmy = (NCHUNKS // NS) + (s < (NCHUNKS % NS)).astype(jnp.int32)
    cN = c * N
    cE = c * EP

    @pl.loop(0, nmy)
    def _chunk(i):
        chunk = s + i * NS
        base = pl.multiple_of(chunk * CH, CH)
        pltpu.sync_copy(col_h.at[pl.ds(base, CH)], coli)

        @pl.loop(0, CH // 16)
        def _adj(j):
            sl = pl.ds(j * 16, 16)
            coli[sl] = coli[sl] + cN

        pltpu.sync_copy(row_h.at[pl.ds(base, CH)], rowi)
        pltpu.sync_copy(nf_h.at[coli], nj)
        pltpu.sync_copy(rw_h.at[pl.ds(pl.multiple_of(cE + base, CH), CH)], rwv)

        @pl.loop(0, CH)
        def _mul(j):
            lo = pl.ds(0, 16)
            hi = pl.ds(16, 16)
            nj[j, lo] = nj[j, lo] * rwv[j, lo]
            nj[j, hi] = nj[j, hi] * rwv[j, hi]

        pltpu.sync_copy(nj, agg_sh.at[rowi], add=True)

    plsc.subcore_barrier()

    @pl.when(s < NS - 1)
    def _():
        pltpu.sync_copy(agg_sh.at[pl.ds(rbase, ROWS_A)],
                        agg_h.at[pl.ds(cN + rbase, ROWS_A)])

    @pl.when(s == NS - 1)
    def _():
        pltpu.sync_copy(agg_sh.at[pl.ds(rbase, ROWS_LAST)],
                        agg_h.at[pl.ds(cN + rbase, ROWS_LAST)])


_sc_messages = functools.partial(
    pl.kernel,
    out_type=jax.ShapeDtypeStruct((2 * N, HH), jnp.float32),
    mesh=_SC_MESH,
    compiler_params=_SC_PARAMS,
    scratch_types=[
        pltpu.VMEM((CH,), jnp.int32),
        pltpu.VMEM((CH,), jnp.int32),
        pltpu.VMEM((CH, HH), jnp.float32),
        pltpu.VMEM((CH, HH), jnp.float32),
        pltpu.VMEM_SHARED((N, HH), jnp.float32),
    ],
)(_sc_messages_body)


def _tc_emb_body(an_ref, emb_ref, nf_ref):
    an = an_ref[...]
    ids = lax.broadcasted_iota(jnp.int32, (BN, NUM_ATOMS), 1)
    oh = (an == ids).astype(jnp.float32)
    nf = jnp.dot(oh, emb_ref[...], preferred_element_type=jnp.float32)
    nf_ref[0, :, :] = nf[:, :HH]
    nf_ref[1, :, :] = nf[:, HH:]


def _tc_radial_body(d2_ref,
                    w0a, b0a, w1a, b1a, w2a, b2a,
                    w0b, b0b, w1b, b1b, w2b, b2b,
                    rw0_ref, rw1_ref):
    d2 = d2_ref[...]
    d = jnp.sqrt(d2)
    th = d * (math.pi / CUTOFF)
    s1 = jnp.sin(th)
    c1 = jnp.cos(th)
    cut = 0.5 * (c1 + 1.0)
    cut = cut * (d < CUTOFF).astype(jnp.float32)
    g = cut / jnp.clip(d, 1e-8, None)
    two_c = 2.0 * c1
    bs = []
    sk_m1 = jnp.zeros_like(s1)
    sk = s1
    for _ in range(NB):
        bs.append(sk * g)
        sk, sk_m1 = two_c * sk - sk_m1, sk
    rbf_t = jnp.stack(bs, axis=0).reshape(NB, BR * 128)
    rbf = jnp.transpose(rbf_t, (1, 0))
    for w0, b0, w1, b1, w2, b2, o_ref in (
        (w0a, b0a, w1a, b1a, w2a, b2a, rw0_ref),
        (w0b, b0b, w1b, b1b, w2b, b2b, rw1_ref),
    ):
        h = _silu(jnp.dot(rbf, w0[...], preferred_element_type=jnp.float32)
                  + b0[...])
        h = _silu(jnp.dot(h, w1[...], preferred_element_type=jnp.float32)
                  + b1[...])
        rw = jnp.dot(h, w2[...], preferred_element_type=jnp.float32) + b2[...]
        o_ref[0, :, :] = rw[:, :HH]
        o_ref[1, :, :] = rw[:, HH:]


def _tc_update_body(nf_ref, agg_ref, wn_ref, wa_ref, b_ref, lnw_ref, lnb_ref,
                    out_ref):
    nf = jnp.concatenate([nf_ref[0], nf_ref[1]], axis=-1)
    ag = jnp.concatenate([agg_ref[0], agg_ref[1]], axis=-1)
    upd = (jnp.dot(nf, wn_ref[...], preferred_element_type=jnp.float32)
           + jnp.dot(ag, wa_ref[...], preferred_element_type=jnp.float32)
           + b_ref[...])
    x = nf + upd
    m = jnp.mean(x, axis=-1, keepdims=True)
    v = jnp.mean((x - m) ** 2, axis=-1, keepdims=True)
    y = (x - m) / jnp.sqrt(v + 1e-5) * lnw_ref[...] + lnb_ref[...]
    out_ref[0, :, :] = y[:, :HH]
    out_ref[1, :, :] = y[:, HH:]


def _tc_readout_body(nf_ref, an_ref, w0_ref, b0_ref, w1_ref, b1_ref, ae_ref,
                     out_ref):
    nf = jnp.concatenate([nf_ref[0], nf_ref[1]], axis=-1)
    t = _silu(jnp.dot(nf, w0_ref[...], preferred_element_type=jnp.float32)
              + b0_ref[...])
    e = jnp.dot(t, w1_ref[...], preferred_element_type=jnp.float32) + b1_ref[...]
    an = an_ref[...]
    ids = lax.broadcasted_iota(jnp.int32, (BN, NUM_ATOMS), 1)
    oh = (an == ids).astype(jnp.float32)
    e = e + jnp.dot(oh, ae_ref[...], preferred_element_type=jnp.float32)

    @pl.when(pl.program_id(0) == 0)
    def _():
        out_ref[...] = jnp.zeros_like(out_ref)

    out_ref[...] = out_ref[...] + jnp.sum(e).reshape(1, 1)


def _full(shape):
    return pl.BlockSpec(shape, lambda i: tuple(0 for _ in shape))


def _tc_emb(an2, emb):
    return pl.pallas_call(
        _tc_emb_body,
        grid=(N // BN,),
        in_specs=[pl.BlockSpec((BN, 1), lambda i: (i, 0)),
                  _full((NUM_ATOMS, H))],
        out_specs=pl.BlockSpec((2, BN, HH), lambda i: (0, i, 0)),
        out_shape=jax.ShapeDtypeStruct((2, N, HH), jnp.float32),
    )(an2, emb)


def _tc_radial(d2, wts):
    return pl.pallas_call(
        _tc_radial_body,
        grid=(ER // BR,),
        in_specs=[pl.BlockSpec((BR, 128), lambda i: (i, 0))]
                 + [_full(w.shape) for w in wts],
        out_specs=[pl.BlockSpec((2, BR * 128, HH), lambda i: (0, i, 0))] * 2,
        out_shape=[jax.ShapeDtypeStruct((2, EP, HH), jnp.float32)] * 2,
    )(d2, *wts)


def _tc_update(nf, agg, wts):
    return pl.pallas_call(
        _tc_update_body,
        grid=(N // BN,),
        in_specs=[pl.BlockSpec((2, BN, HH), lambda i: (0, i, 0))] * 2
                 + [_full(w.shape) for w in wts],
        out_specs=pl.BlockSpec((2, BN, HH), lambda i: (0, i, 0)),
        out_shape=jax.ShapeDtypeStruct((2, N, HH), jnp.float32),
    )(nf, agg, *wts)


def _tc_readout(nf, an2, wts):
    return pl.pallas_call(
        _tc_readout_body,
        grid=(N // BN,),
        in_specs=[pl.BlockSpec((2, BN, HH), lambda i: (0, i, 0)),
                  pl.BlockSpec((BN, 1), lambda i: (i, 0))]
                 + [_full(w.shape) for w in wts],
        out_specs=pl.BlockSpec((1, 1), lambda i: (0, 0)),
        out_shape=jax.ShapeDtypeStruct((1, 1), jnp.float32),
    )(nf, an2, *wts)


def kernel(atomic_numbers, pos, edge_index, params):
    row = edge_index[0]
    col = edge_index[1]
    px, py, pz = pos[:, 0], pos[:, 1], pos[:, 2]
    an2 = atomic_numbers.reshape(N, 1)

    d2 = _sc_edge_d2(row, col, px, py, pz)
    nf = _tc_emb(an2, params["emb"])

    rwts = []
    for lp in params["layers"]:
        w2f = lp["rn2"]["w"].reshape(H, H, 3).sum(-1)
        b2f = lp["rn2"]["b"].reshape(H, 3).sum(-1)
        rwts += [lp["rn0"]["w"], lp["rn0"]["b"].reshape(1, H),
                 lp["rn1"]["w"], lp["rn1"]["b"].reshape(1, H),
                 w2f, b2f.reshape(1, H)]
    rw0, rw1 = _tc_radial(d2.reshape(ER, 128), rwts)

    zer = jnp.zeros((ROWS_A, HH), jnp.float32)
    for li, lp in enumerate(params["layers"]):
        rw = (rw0, rw1)[li]
        agg = _sc_messages(row, col, nf.reshape(2 * N, HH),
                           rw.reshape(2 * EP, HH), zer)
        uwts = [lp["lin"]["w"][:H], lp["lin"]["w"][H:],
                lp["lin"]["b"].reshape(1, H),
                lp["ln_w"].reshape(1, H), lp["ln_b"].reshape(1, H)]
        nf = _tc_update(nf, agg.reshape(2, N, HH), uwts)

    owts = [params["ro0"]["w"], params["ro0"]["b"].reshape(1, H),
            params["ro1"]["w"], params["ro1"]["b"].reshape(1, 1),
            params["atomic_e"]]
    tot = _tc_readout(nf, an2, owts)
    return tot[0, 0] * params["scale"] + params["shift"]


# speedup vs baseline: 4.8940x; 1.6688x over previous
import functools
import math

import jax
import jax.numpy as jnp
from jax import lax
from jax.experimental import pallas as pl
from jax.experimental.pallas import tpu as pltpu
from jax.experimental.pallas import tpu_sc as plsc

N = 50000
E = 800000
H = 64
HH = H // 2
NB = 8
NUM_ATOMS = 100
CUTOFF = 5.0

NC = 2
NS = 16
NW = NC * NS
CH = 128
NCHUNKS = E // CH
ROWS_A = 3128
ROWS_LAST = N - (NS - 1) * ROWS_A

BN = 2000
BR = 64
EP = 819200
ER = EP // 128

_SC_MESH = plsc.VectorSubcoreMesh(
    core_axis_name="c", subcore_axis_name="s", num_cores=NC, num_subcores=NS
)


def _silu(x):
    return x * (1.0 / (1.0 + jnp.exp(-x)))


_SC_PARAMS = pltpu.CompilerParams(use_tc_tiling_on_sc=False)
_D2_SLOTS = NCHUNKS // NW + 3


def _sc_edge_d2_body(row_h, col_h, px_h, py_h, pz_h, d2_h, *sc):
    c = lax.axis_index("c")
    s = lax.axis_index("s")
    wid = s * NC + c
    names = ("rowi", "coli", "g", "d2v", "sidx", "sg", "swr")
    A = dict(zip(names, sc[0:7]))
    B = dict(zip(names, sc[7:14]))

    def valid(j):
        return jnp.logical_and(j >= 0, wid + j * NW < NCHUNKS)

    def base_of(j):
        cid = wid + j * NW
        cid = jnp.where(valid(j), cid, 0)
        return pl.multiple_of(cid * CH, CH)

    def issue_idx(j, X):
        bs = base_of(j)
        pltpu.async_copy(row_h.at[pl.ds(bs, CH)], X["rowi"], X["sidx"])
        pltpu.async_copy(col_h.at[pl.ds(bs, CH)], X["coli"], X["sidx"])

    def wait_idx(X):
        pltpu.make_async_copy(row_h.at[pl.ds(0, CH)], X["rowi"], X["sidx"]).wait()
        pltpu.make_async_copy(col_h.at[pl.ds(0, CH)], X["coli"], X["sidx"]).wait()

    def issue_gathers(X):
        for k, (tb, ib) in enumerate(((px_h, "rowi"), (py_h, "rowi"),
                                      (pz_h, "rowi"), (px_h, "coli"),
                                      (py_h, "coli"), (pz_h, "coli"))):
            pltpu.async_copy(tb.at[X[ib]], X["g"].at[k], X["sg"])

    def wait_gathers(X):
        for k in range(6):
            pltpu.make_async_copy(px_h.at[X["rowi"]], X["g"].at[k],
                                  X["sg"]).wait()

    def compute_and_write(j, X):
        g = X["g"]
        d2v = X["d2v"]
        for t in range(CH // 16):
            sl = pl.ds(t * 16, 16)
            dx = g[0, sl] - g[3, sl]
            dy = g[1, sl] - g[4, sl]
            dz = g[2, sl] - g[5, sl]
            d2v[sl] = dx * dx + dy * dy + dz * dz
        pltpu.async_copy(d2v, d2_h.at[pl.ds(base_of(j), CH)], X["swr"])

    def wait_write(X):
        pltpu.make_async_copy(X["d2v"], d2_h.at[pl.ds(0, CH)], X["swr"]).wait()

    def half(i, CUR, NXT):
        @pl.when(valid(i))
        def _():
            wait_idx(CUR)

        @pl.when(valid(i - 2))
        def _():
            wait_write(CUR)

        @pl.when(valid(i))
        def _():
            issue_gathers(CUR)

        @pl.when(valid(i - 1))
        def _():
            wait_gathers(NXT)
            compute_and_write(i - 1, NXT)

        @pl.when(valid(i + 1))
        def _():
            issue_idx(i + 1, NXT)

    issue_idx(0, A)

    @pl.loop(0, _D2_SLOTS // 2)
    def _pair(k):
        half(2 * k, A, B)
        half(2 * k + 1, B, A)

    @pl.when(valid(_D2_SLOTS - 2))
    def _():
        wait_write(A)

    @pl.when(valid(_D2_SLOTS - 1))
    def _():
        wait_write(B)


def _d2_scratch_set():
    return [
        pltpu.VMEM((CH,), jnp.int32),
        pltpu.VMEM((CH,), jnp.int32),
        pltpu.VMEM((6, CH), jnp.float32),
        pltpu.VMEM((CH,), jnp.float32),
        pltpu.SemaphoreType.DMA,
        pltpu.SemaphoreType.DMA,
        pltpu.SemaphoreType.DMA,
    ]


_sc_edge_d2 = functools.partial(
    pl.kernel,
    out_type=jax.ShapeDtypeStruct((EP,), jnp.float32),
    mesh=_SC_MESH,
    compiler_params=_SC_PARAMS,
    scratch_types=_d2_scratch_set() + _d2_scratch_set(),
)(_sc_edge_d2_body)


_MSG_SLOTS = NCHUNKS // NS + 4


def _sc_messages_body(row_h, col_h, nf_h, rw_h, zer_h, agg_h, *sc):
    c = lax.axis_index("c")
    s = lax.axis_index("s")
    names = ("rowi", "rowsc", "coli", "nj", "rwv", "sidx", "sg", "sw", "ssc")
    A = dict(zip(names, sc[0:9]))
    B = dict(zip(names, sc[9:18]))
    agg_sh = sc[18]

    rbase = pl.multiple_of(s * ROWS_A, 8)

    @pl.when(s < NS - 1)
    def _():
        pltpu.sync_copy(zer_h, agg_sh.at[pl.ds(rbase, ROWS_A)])

    @pl.when(s == NS - 1)
    def _():
        pltpu.sync_copy(zer_h.at[pl.ds(0, ROWS_LAST)],
                        agg_sh.at[pl.ds(rbase, ROWS_LAST)])

    plsc.subcore_barrier()

    cN = c * N
    cE = c * EP

    def valid(j):
        return jnp.logical_and(j >= 0, s + j * NS < NCHUNKS)

    def base_of(j):
        cid = s + j * NS
        cid = jnp.where(valid(j), cid, 0)
        return pl.multiple_of(cid * CH, CH)

    def issue_idx(j, X):
        bs = base_of(j)
        pltpu.async_copy(row_h.at[pl.ds(bs, CH)], X["rowi"], X["sidx"])
        pltpu.async_copy(col_h.at[pl.ds(bs, CH)], X["coli"], X["sidx"])

    def wait_idx(X):
        pltpu.make_async_copy(row_h.at[pl.ds(0, CH)], X["rowi"], X["sidx"]).wait()
        pltpu.make_async_copy(col_h.at[pl.ds(0, CH)], X["coli"], X["sidx"]).wait()

    def issue_loads(j, X):
        for t in range(CH // 16):
            sl = pl.ds(t * 16, 16)
            X["coli"][sl] = X["coli"][sl] + cN
        pltpu.async_copy(nf_h.at[X["coli"]], X["nj"], X["sg"])
        pltpu.async_copy(rw_h.at[pl.ds(pl.multiple_of(cE + base_of(j), CH), CH)],
                         X["rwv"], X["sw"])

    def wait_loads(X):
        pltpu.make_async_copy(nf_h.at[X["coli"]], X["nj"], X["sg"]).wait()
        pltpu.make_async_copy(rw_h.at[pl.ds(0, CH)], X["rwv"], X["sw"]).wait()

    def mult_scatter(X):
        nj = X["nj"]
        rwv = X["rwv"]
        lo = pl.ds(0, 16)
        hi = pl.ds(16, 16)

        @pl.loop(0, CH, unroll=4)
        def _mul(j):
            nj[j, lo] = nj[j, lo] * rwv[j, lo]
            nj[j, hi] = nj[j, hi] * rwv[j, hi]

        for t in range(CH // 16):
            sl = pl.ds(t * 16, 16)
            X["rowsc"][sl] = X["rowi"][sl]
        pltpu.async_copy(nj, agg_sh.at[X["rowsc"]], X["ssc"], add=True)

    def wait_scatter(X):
        pltpu.make_async_copy(X["nj"], agg_sh.at[X["rowsc"]], X["ssc"]).wait()

    def half(i, CUR, NXT):
        @pl.when(valid(i))
        def _():
            wait_idx(CUR)

        @pl.when(valid(i - 2))
        def _():
            wait_scatter(CUR)

        @pl.when(valid(i))
        def _():
            issue_loads(i, CUR)

        @pl.when(valid(i - 1))
        def _():
            wait_loads(NXT)
            mult_scatter(NXT)

        @pl.when(valid(i + 1))
        def _():
            issue_idx(i + 1, NXT)

    issue_idx(0, A)

    @pl.loop(0, _MSG_SLOTS // 2)
    def _pair(k):
        half(2 * k, A, B)
        half(2 * k + 1, B, A)

    plsc.subcore_barrier()

    @pl.when(s < NS - 1)
    def _():
        pltpu.sync_copy(agg_sh.at[pl.ds(rbase, ROWS_A)],
                        agg_h.at[pl.ds(cN + rbase, ROWS_A)])

    @pl.when(s == NS - 1)
    def _():
        pltpu.sync_copy(agg_sh.at[pl.ds(rbase, ROWS_LAST)],
                        agg_h.at[pl.ds(cN + rbase, ROWS_LAST)])


def _msg_scratch_set():
    return [
        pltpu.VMEM((CH,), jnp.int32),
        pltpu.VMEM((CH,), jnp.int32),
        pltpu.VMEM((CH,), jnp.int32),
        pltpu.VMEM((CH, HH), jnp.float32),
        pltpu.VMEM((CH, HH), jnp.float32),
        pltpu.SemaphoreType.DMA,
        pltpu.SemaphoreType.DMA,
        pltpu.SemaphoreType.DMA,
        pltpu.SemaphoreType.DMA,
    ]


_sc_messages = functools.partial(
    pl.kernel,
    out_type=jax.ShapeDtypeStruct((2 * N, HH), jnp.float32),
    mesh=_SC_MESH,
    compiler_params=_SC_PARAMS,
    scratch_types=_msg_scratch_set() + _msg_scratch_set()
                  + [pltpu.VMEM_SHARED((N, HH), jnp.float32)],
)(_sc_messages_body)


def _tc_emb_body(an_ref, emb_ref, nf_ref):
    an = an_ref[...]
    ids = lax.broadcasted_iota(jnp.int32, (BN, NUM_ATOMS), 1)
    oh = (an == ids).astype(jnp.float32)
    nf = jnp.dot(oh, emb_ref[...], preferred_element_type=jnp.float32)
    nf_ref[0, :, :] = nf[:, :HH]
    nf_ref[1, :, :] = nf[:, HH:]


def _tc_radial_body(d2_ref,
                    w0a, b0a, w1a, b1a, w2a, b2a,
                    w0b, b0b, w1b, b1b, w2b, b2b,
                    rw0_ref, rw1_ref):
    d2 = d2_ref[...]
    d = jnp.sqrt(d2)
    th = d * (math.pi / CUTOFF)
    s1 = jnp.sin(th)
    c1 = jnp.cos(th)
    cut = 0.5 * (c1 + 1.0)
    cut = cut * (d < CUTOFF).astype(jnp.float32)
    g = cut / jnp.clip(d, 1e-8, None)
    two_c = 2.0 * c1
    bs = []
    sk_m1 = jnp.zeros_like(s1)
    sk = s1
    for _ in range(NB):
        bs.append(sk * g)
        sk, sk_m1 = two_c * sk - sk_m1, sk
    rbf_t = jnp.stack(bs, axis=0).reshape(NB, BR * 128)
    rbf = jnp.transpose(rbf_t, (1, 0))
    for w0, b0, w1, b1, w2, b2, o_ref in (
        (w0a, b0a, w1a, b1a, w2a, b2a, rw0_ref),
        (w0b, b0b, w1b, b1b, w2b, b2b, rw1_ref),
    ):
        h = _silu(jnp.dot(rbf, w0[...], preferred_element_type=jnp.float32)
                  + b0[...])
        h = _silu(jnp.dot(h, w1[...], preferred_element_type=jnp.float32)
                  + b1[...])
        rw = jnp.dot(h, w2[...], preferred_element_type=jnp.float32) + b2[...]
        o_ref[0, :, :] = rw[:, :HH]
        o_ref[1, :, :] = rw[:, HH:]


def _tc_update_body(nf_ref, agg_ref, wn_ref, wa_ref, b_ref, lnw_ref, lnb_ref,
                    out_ref):
    nf = jnp.concatenate([nf_ref[0], nf_ref[1]], axis=-1)
    ag = jnp.concatenate([agg_ref[0], agg_ref[1]], axis=-1)
    upd = (jnp.dot(nf, wn_ref[...], preferred_element_type=jnp.float32)
           + jnp.dot(ag, wa_ref[...], preferred_element_type=jnp.float32)
           + b_ref[...])
    x = nf + upd
    m = jnp.mean(x, axis=-1, keepdims=True)
    v = jnp.mean((x - m) ** 2, axis=-1, keepdims=True)
    y = (x - m) / jnp.sqrt(v + 1e-5) * lnw_ref[...] + lnb_ref[...]
    out_ref[0, :, :] = y[:, :HH]
    out_ref[1, :, :] = y[:, HH:]


def _tc_readout_body(nf_ref, an_ref, w0_ref, b0_ref, w1_ref, b1_ref, ae_ref,
                     out_ref):
    nf = jnp.concatenate([nf_ref[0], nf_ref[1]], axis=-1)
    t = _silu(jnp.dot(nf, w0_ref[...], preferred_element_type=jnp.float32)
              + b0_ref[...])
    e = jnp.dot(t, w1_ref[...], preferred_element_type=jnp.float32) + b1_ref[...]
    an = an_ref[...]
    ids = lax.broadcasted_iota(jnp.int32, (BN, NUM_ATOMS), 1)
    oh = (an == ids).astype(jnp.float32)
    e = e + jnp.dot(oh, ae_ref[...], preferred_element_type=jnp.float32)

    @pl.when(pl.program_id(0) == 0)
    def _():
        out_ref[...] = jnp.zeros_like(out_ref)

    out_ref[...] = out_ref[...] + jnp.sum(e).reshape(1, 1)


def _full(shape):
    return pl.BlockSpec(shape, lambda i: tuple(0 for _ in shape))


def _tc_emb(an2, emb):
    return pl.pallas_call(
        _tc_emb_body,
        grid=(N // BN,),
        in_specs=[pl.BlockSpec((BN, 1), lambda i: (i, 0)),
                  _full((NUM_ATOMS, H))],
        out_specs=pl.BlockSpec((2, BN, HH), lambda i: (0, i, 0)),
        out_shape=jax.ShapeDtypeStruct((2, N, HH), jnp.float32),
    )(an2, emb)


def _tc_radial(d2, wts):
    return pl.pallas_call(
        _tc_radial_body,
        grid=(ER // BR,),
        in_specs=[pl.BlockSpec((BR, 128), lambda i: (i, 0))]
                 + [_full(w.shape) for w in wts],
        out_specs=[pl.BlockSpec((2, BR * 128, HH), lambda i: (0, i, 0))] * 2,
        out_shape=[jax.ShapeDtypeStruct((2, EP, HH), jnp.float32)] * 2,
    )(d2, *wts)


def _tc_update(nf, agg, wts):
    return pl.pallas_call(
        _tc_update_body,
        grid=(N // BN,),
        in_specs=[pl.BlockSpec((2, BN, HH), lambda i: (0, i, 0))] * 2
                 + [_full(w.shape) for w in wts],
        out_specs=pl.BlockSpec((2, BN, HH), lambda i: (0, i, 0)),
        out_shape=jax.ShapeDtypeStruct((2, N, HH), jnp.float32),
    )(nf, agg, *wts)


def _tc_readout(nf, an2, wts):
    return pl.pallas_call(
        _tc_readout_body,
        grid=(N // BN,),
        in_specs=[pl.BlockSpec((2, BN, HH), lambda i: (0, i, 0)),
                  pl.BlockSpec((BN, 1), lambda i: (i, 0))]
                 + [_full(w.shape) for w in wts],
        out_specs=pl.BlockSpec((1, 1), lambda i: (0, 0)),
        out_shape=jax.ShapeDtypeStruct((1, 1), jnp.float32),
    )(nf, an2, *wts)


def kernel(atomic_numbers, pos, edge_index, params):
    row = edge_index[0]
    col = edge_index[1]
    px, py, pz = pos[:, 0], pos[:, 1], pos[:, 2]
    an2 = atomic_numbers.reshape(N, 1)

    d2 = _sc_edge_d2(row, col, px, py, pz)
    nf = _tc_emb(an2, params["emb"])

    rwts = []
    for lp in params["layers"]:
        w2f = lp["rn2"]["w"].reshape(H, H, 3).sum(-1)
        b2f = lp["rn2"]["b"].reshape(H, 3).sum(-1)
        rwts += [lp["rn0"]["w"], lp["rn0"]["b"].reshape(1, H),
                 lp["rn1"]["w"], lp["rn1"]["b"].reshape(1, H),
                 w2f, b2f.reshape(1, H)]
    rw0, rw1 = _tc_radial(d2.reshape(ER, 128), rwts)

    zer = jnp.zeros((ROWS_A, HH), jnp.float32)
    for li, lp in enumerate(params["layers"]):
        rw = (rw0, rw1)[li]
        agg = _sc_messages(row, col, nf.reshape(2 * N, HH),
                           rw.reshape(2 * EP, HH), zer)
        uwts = [lp["lin"]["w"][:H], lp["lin"]["w"][H:],
                lp["lin"]["b"].reshape(1, H),
                lp["ln_w"].reshape(1, H), lp["ln_b"].reshape(1, H)]
        nf = _tc_update(nf, agg.reshape(2, N, HH), uwts)

    owts = [params["ro0"]["w"], params["ro0"]["b"].reshape(1, H),
            params["ro1"]["w"], params["ro1"]["b"].reshape(1, 1),
            params["atomic_e"]]
    tot = _tc_readout(nf, an2, owts)
    return tot[0, 0] * params["scale"] + params["shift"]


# speedup vs baseline: 4.9397x; 1.0093x over previous
import functools
import math

import jax
import jax.numpy as jnp
from jax import lax
from jax.experimental import pallas as pl
from jax.experimental.pallas import tpu as pltpu
from jax.experimental.pallas import tpu_sc as plsc

N = 50000
E = 800000
H = 64
HH = H // 2
NB = 8
NUM_ATOMS = 100
CUTOFF = 5.0

NC = 2
NS = 16
NW = NC * NS
CH = 128
NCHUNKS = E // CH
ROWS_A = 3128
ROWS_LAST = N - (NS - 1) * ROWS_A

BN = 2000
BR = 64
EP = 819200
ER = EP // 128

_SC_MESH = plsc.VectorSubcoreMesh(
    core_axis_name="c", subcore_axis_name="s", num_cores=NC, num_subcores=NS
)


def _silu(x):
    return x * (1.0 / (1.0 + jnp.exp(-x)))


_SC_PARAMS = pltpu.CompilerParams(use_tc_tiling_on_sc=False)
_D2_SLOTS = NCHUNKS // NW + 3


def _sc_edge_d2_body(ei_h, px_h, py_h, pz_h, d2_h, *sc):
    c = lax.axis_index("c")
    s = lax.axis_index("s")
    wid = s * NC + c
    names = ("rowi", "coli", "g", "d2v", "sidx", "sg", "swr")
    A = dict(zip(names, sc[0:7]))
    B = dict(zip(names, sc[7:14]))

    def valid(j):
        return jnp.logical_and(j >= 0, wid + j * NW < NCHUNKS)

    def chunk_of(j):
        cid = wid + j * NW
        return jnp.where(valid(j), cid, 0)

    def base_of(j):
        return pl.multiple_of(chunk_of(j) * CH, CH)

    def issue_idx(j, X):
        bs = base_of(j)
        pltpu.async_copy(ei_h.at[0, pl.ds(bs, CH)], X["rowi"], X["sidx"])
        pltpu.async_copy(ei_h.at[1, pl.ds(bs, CH)], X["coli"], X["sidx"])

    def wait_idx(X):
        pltpu.make_async_copy(ei_h.at[0, pl.ds(0, CH)], X["rowi"], X["sidx"]).wait()
        pltpu.make_async_copy(ei_h.at[1, pl.ds(0, CH)], X["coli"], X["sidx"]).wait()

    def issue_gathers(X):
        for k, (tb, ib) in enumerate(((px_h, "rowi"), (py_h, "rowi"),
                                      (pz_h, "rowi"), (px_h, "coli"),
                                      (py_h, "coli"), (pz_h, "coli"))):
            pltpu.async_copy(tb.at[X[ib]], X["g"].at[k], X["sg"])

    def wait_gathers(X):
        for k in range(6):
            pltpu.make_async_copy(px_h.at[X["rowi"]], X["g"].at[k],
                                  X["sg"]).wait()

    def compute_and_write(j, X):
        g = X["g"]
        d2v = X["d2v"]
        for t in range(CH // 16):
            sl = pl.ds(t * 16, 16)
            dx = g[0, sl] - g[3, sl]
            dy = g[1, sl] - g[4, sl]
            dz = g[2, sl] - g[5, sl]
            d2v[sl] = dx * dx + dy * dy + dz * dz
        pltpu.async_copy(d2v, d2_h.at[chunk_of(j)], X["swr"])

    def wait_write(X):
        pltpu.make_async_copy(X["d2v"], d2_h.at[0], X["swr"]).wait()

    def half(i, CUR, NXT):
        @pl.when(valid(i))
        def _():
            wait_idx(CUR)

        @pl.when(valid(i - 2))
        def _():
            wait_write(CUR)

        @pl.when(valid(i))
        def _():
            issue_gathers(CUR)

        @pl.when(valid(i - 1))
        def _():
            wait_gathers(NXT)
            compute_and_write(i - 1, NXT)

        @pl.when(valid(i + 1))
        def _():
            issue_idx(i + 1, NXT)

    issue_idx(0, A)

    @pl.loop(0, _D2_SLOTS // 2)
    def _pair(k):
        half(2 * k, A, B)
        half(2 * k + 1, B, A)

    @pl.when(valid(_D2_SLOTS - 2))
    def _():
        wait_write(A)

    @pl.when(valid(_D2_SLOTS - 1))
    def _():
        wait_write(B)


def _d2_scratch_set():
    return [
        pltpu.VMEM((CH,), jnp.int32),
        pltpu.VMEM((CH,), jnp.int32),
        pltpu.VMEM((6, CH), jnp.float32),
        pltpu.VMEM((CH,), jnp.float32),
        pltpu.SemaphoreType.DMA,
        pltpu.SemaphoreType.DMA,
        pltpu.SemaphoreType.DMA,
    ]


_sc_edge_d2 = functools.partial(
    pl.kernel,
    out_type=jax.ShapeDtypeStruct((ER, 128), jnp.float32),
    mesh=_SC_MESH,
    compiler_params=_SC_PARAMS,
    scratch_types=_d2_scratch_set() + _d2_scratch_set(),
)(_sc_edge_d2_body)


_MSG_SLOTS = NCHUNKS // NS + 4


def _sc_messages_body(ei_h, nf_h, rw_h, zer_h, agg_h, *sc):
    c = lax.axis_index("c")
    s = lax.axis_index("s")
    names = ("rowi", "rowsc", "coli", "nj", "rwv", "sidx", "sg", "sw", "ssc")
    A = dict(zip(names, sc[0:9]))
    B = dict(zip(names, sc[9:18]))
    agg_sh = sc[18]
    nf_c = nf_h.at[c]
    rw_c = rw_h.at[c]
    agg_c = agg_h.at[c]

    rbase = pl.multiple_of(s * ROWS_A, 8)

    @pl.when(s < NS - 1)
    def _():
        pltpu.sync_copy(zer_h, agg_sh.at[pl.ds(rbase, ROWS_A)])

    @pl.when(s == NS - 1)
    def _():
        pltpu.sync_copy(zer_h.at[pl.ds(0, ROWS_LAST)],
                        agg_sh.at[pl.ds(rbase, ROWS_LAST)])

    plsc.subcore_barrier()

    def valid(j):
        return jnp.logical_and(j >= 0, s + j * NS < NCHUNKS)

    def base_of(j):
        cid = s + j * NS
        cid = jnp.where(valid(j), cid, 0)
        return pl.multiple_of(cid * CH, CH)

    def issue_idx(j, X):
        bs = base_of(j)
        pltpu.async_copy(ei_h.at[0, pl.ds(bs, CH)], X["rowi"], X["sidx"])
        pltpu.async_copy(ei_h.at[1, pl.ds(bs, CH)], X["coli"], X["sidx"])

    def wait_idx(X):
        pltpu.make_async_copy(ei_h.at[0, pl.ds(0, CH)], X["rowi"], X["sidx"]).wait()
        pltpu.make_async_copy(ei_h.at[1, pl.ds(0, CH)], X["coli"], X["sidx"]).wait()

    def issue_loads(j, X):
        pltpu.async_copy(nf_c.at[X["coli"]], X["nj"], X["sg"])
        pltpu.async_copy(rw_c.at[pl.ds(base_of(j), CH)], X["rwv"], X["sw"])

    def wait_loads(X):
        pltpu.make_async_copy(nf_c.at[X["coli"]], X["nj"], X["sg"]).wait()
        pltpu.make_async_copy(rw_c.at[pl.ds(0, CH)], X["rwv"], X["sw"]).wait()

    def mult_scatter(X):
        nj = X["nj"]
        rwv = X["rwv"]
        lo = pl.ds(0, 16)
        hi = pl.ds(16, 16)

        @pl.loop(0, CH, unroll=4)
        def _mul(j):
            nj[j, lo] = nj[j, lo] * rwv[j, lo]
            nj[j, hi] = nj[j, hi] * rwv[j, hi]

        for t in range(CH // 16):
            sl = pl.ds(t * 16, 16)
            X["rowsc"][sl] = X["rowi"][sl]
        pltpu.async_copy(nj, agg_sh.at[X["rowsc"]], X["ssc"], add=True)

    def wait_scatter(X):
        pltpu.make_async_copy(X["nj"], agg_sh.at[X["rowsc"]], X["ssc"]).wait()

    def half(i, CUR, NXT):
        @pl.when(valid(i))
        def _():
            wait_idx(CUR)

        @pl.when(valid(i - 2))
        def _():
            wait_scatter(CUR)

        @pl.when(valid(i))
        def _():
            issue_loads(i, CUR)

        @pl.when(valid(i - 1))
        def _():
            wait_loads(NXT)
            mult_scatter(NXT)

        @pl.when(valid(i + 1))
        def _():
            issue_idx(i + 1, NXT)

    issue_idx(0, A)

    @pl.loop(0, _MSG_SLOTS // 2)
    def _pair(k):
        half(2 * k, A, B)
        half(2 * k + 1, B, A)

    plsc.subcore_barrier()

    @pl.when(s < NS - 1)
    def _():
        pltpu.sync_copy(agg_sh.at[pl.ds(rbase, ROWS_A)],
                        agg_c.at[pl.ds(rbase, ROWS_A)])

    @pl.when(s == NS - 1)
    def _():
        pltpu.sync_copy(agg_sh.at[pl.ds(rbase, ROWS_LAST)],
                        agg_c.at[pl.ds(rbase, ROWS_LAST)])


def _msg_scratch_set():
    return [
        pltpu.VMEM((CH,), jnp.int32),
        pltpu.VMEM((CH,), jnp.int32),
        pltpu.VMEM((CH,), jnp.int32),
        pltpu.VMEM((CH, HH), jnp.float32),
        pltpu.VMEM((CH, HH), jnp.float32),
        pltpu.SemaphoreType.DMA,
        pltpu.SemaphoreType.DMA,
        pltpu.SemaphoreType.DMA,
        pltpu.SemaphoreType.DMA,
    ]


_sc_messages = functools.partial(
    pl.kernel,
    out_type=jax.ShapeDtypeStruct((2, N, HH), jnp.float32),
    mesh=_SC_MESH,
    compiler_params=_SC_PARAMS,
    scratch_types=_msg_scratch_set() + _msg_scratch_set()
                  + [pltpu.VMEM_SHARED((N, HH), jnp.float32)],
)(_sc_messages_body)


def _tc_emb_body(an_ref, emb_ref, nf_ref):
    an = an_ref[...]
    ids = lax.broadcasted_iota(jnp.int32, (BN, NUM_ATOMS), 1)
    oh = (an == ids).astype(jnp.float32)
    nf = jnp.dot(oh, emb_ref[...], preferred_element_type=jnp.float32)
    nf_ref[0, :, :] = nf[:, :HH]
    nf_ref[1, :, :] = nf[:, HH:]


def _tc_radial_body(d2_ref,
                    w0a, b0a, w1a, b1a, w2a, b2a,
                    w0b, b0b, w1b, b1b, w2b, b2b,
                    rw0_ref, rw1_ref):
    d2 = d2_ref[...]
    d = jnp.sqrt(d2)
    th = d * (math.pi / CUTOFF)
    s1 = jnp.sin(th)
    c1 = jnp.cos(th)
    cut = 0.5 * (c1 + 1.0)
    cut = cut * (d < CUTOFF).astype(jnp.float32)
    g = cut / jnp.clip(d, 1e-8, None)
    two_c = 2.0 * c1
    bs = []
    sk_m1 = jnp.zeros_like(s1)
    sk = s1
    for _ in range(NB):
        bs.append(sk * g)
        sk, sk_m1 = two_c * sk - sk_m1, sk
    rbf_t = jnp.stack(bs, axis=0).reshape(NB, BR * 128)
    rbf = jnp.transpose(rbf_t, (1, 0))
    for w0, b0, w1, b1, w2, b2, o_ref in (
        (w0a, b0a, w1a, b1a, w2a, b2a, rw0_ref),
        (w0b, b0b, w1b, b1b, w2b, b2b, rw1_ref),
    ):
        h = _silu(jnp.dot(rbf, w0[...], preferred_element_type=jnp.float32)
                  + b0[...])
        h = _silu(jnp.dot(h, w1[...], preferred_element_type=jnp.float32)
                  + b1[...])
        rw = jnp.dot(h, w2[...], preferred_element_type=jnp.float32) + b2[...]
        o_ref[0, :, :] = rw[:, :HH]
        o_ref[1, :, :] = rw[:, HH:]


def _tc_update_body(nf_ref, agg_ref, wn_ref, wa_ref, b_ref, lnw_ref, lnb_ref,
                    out_ref):
    nf = jnp.concatenate([nf_ref[0], nf_ref[1]], axis=-1)
    ag = jnp.concatenate([agg_ref[0], agg_ref[1]], axis=-1)
    upd = (jnp.dot(nf, wn_ref[...], preferred_element_type=jnp.float32)
           + jnp.dot(ag, wa_ref[...], preferred_element_type=jnp.float32)
           + b_ref[...])
    x = nf + upd
    m = jnp.mean(x, axis=-1, keepdims=True)
    v = jnp.mean((x - m) ** 2, axis=-1, keepdims=True)
    y = (x - m) / jnp.sqrt(v + 1e-5) * lnw_ref[...] + lnb_ref[...]
    out_ref[0, :, :] = y[:, :HH]
    out_ref[1, :, :] = y[:, HH:]


def _tc_readout_body(nf_ref, an_ref, w0_ref, b0_ref, w1_ref, b1_ref, ae_ref,
                     out_ref):
    nf = jnp.concatenate([nf_ref[0], nf_ref[1]], axis=-1)
    t = _silu(jnp.dot(nf, w0_ref[...], preferred_element_type=jnp.float32)
              + b0_ref[...])
    e = jnp.dot(t, w1_ref[...], preferred_element_type=jnp.float32) + b1_ref[...]
    an = an_ref[...]
    ids = lax.broadcasted_iota(jnp.int32, (BN, NUM_ATOMS), 1)
    oh = (an == ids).astype(jnp.float32)
    e = e + jnp.dot(oh, ae_ref[...], preferred_element_type=jnp.float32)

    @pl.when(pl.program_id(0) == 0)
    def _():
        out_ref[...] = jnp.zeros_like(out_ref)

    out_ref[...] = out_ref[...] + jnp.sum(e).reshape(1, 1)


def _full(shape):
    return pl.BlockSpec(shape, lambda i: tuple(0 for _ in shape))


def _tc_emb(an2, emb):
    return pl.pallas_call(
        _tc_emb_body,
        grid=(N // BN,),
        in_specs=[pl.BlockSpec((BN, 1), lambda i: (i, 0)),
                  _full((NUM_ATOMS, H))],
        out_specs=pl.BlockSpec((2, BN, HH), lambda i: (0, i, 0)),
        out_shape=jax.ShapeDtypeStruct((2, N, HH), jnp.float32),
    )(an2, emb)


def _tc_radial(d2, wts):
    return pl.pallas_call(
        _tc_radial_body,
        grid=(ER // BR,),
        in_specs=[pl.BlockSpec((BR, 128), lambda i: (i, 0))]
                 + [_full(w.shape) for w in wts],
        out_specs=[pl.BlockSpec((2, BR * 128, HH), lambda i: (0, i, 0))] * 2,
        out_shape=[jax.ShapeDtypeStruct((2, EP, HH), jnp.float32)] * 2,
    )(d2, *wts)


def _tc_update(nf, agg, wts):
    return pl.pallas_call(
        _tc_update_body,
        grid=(N // BN,),
        in_specs=[pl.BlockSpec((2, BN, HH), lambda i: (0, i, 0))] * 2
                 + [_full(w.shape) for w in wts],
        out_specs=pl.BlockSpec((2, BN, HH), lambda i: (0, i, 0)),
        out_shape=jax.ShapeDtypeStruct((2, N, HH), jnp.float32),
    )(nf, agg, *wts)


def _tc_readout(nf, an2, wts):
    return pl.pallas_call(
        _tc_readout_body,
        grid=(N // BN,),
        in_specs=[pl.BlockSpec((2, BN, HH), lambda i: (0, i, 0)),
                  pl.BlockSpec((BN, 1), lambda i: (i, 0))]
                 + [_full(w.shape) for w in wts],
        out_specs=pl.BlockSpec((1, 1), lambda i: (0, 0)),
        out_shape=jax.ShapeDtypeStruct((1, 1), jnp.float32),
    )(nf, an2, *wts)


def kernel(atomic_numbers, pos, edge_index, params):
    px, py, pz = pos[:, 0], pos[:, 1], pos[:, 2]
    an2 = atomic_numbers.reshape(N, 1)

    d2 = _sc_edge_d2(edge_index, px, py, pz)
    nf = _tc_emb(an2, params["emb"])

    rwts = []
    for lp in params["layers"]:
        w2f = lp["rn2"]["w"].reshape(H, H, 3).sum(-1)
        b2f = lp["rn2"]["b"].reshape(H, 3).sum(-1)
        rwts += [lp["rn0"]["w"], lp["rn0"]["b"].reshape(1, H),
                 lp["rn1"]["w"], lp["rn1"]["b"].reshape(1, H),
                 w2f, b2f.reshape(1, H)]
    rw0, rw1 = _tc_radial(d2, rwts)

    zer = jnp.zeros((ROWS_A, HH), jnp.float32)
    for li, lp in enumerate(params["layers"]):
        rw = (rw0, rw1)[li]
        agg = _sc_messages(edge_index, nf, rw, zer)
        uwts = [lp["lin"]["w"][:H], lp["lin"]["w"][H:],
                lp["lin"]["b"].reshape(1, H),
                lp["ln_w"].reshape(1, H), lp["ln_b"].reshape(1, H)]
        nf = _tc_update(nf, agg, uwts)

    owts = [params["ro0"]["w"], params["ro0"]["b"].reshape(1, H),
            params["ro1"]["w"], params["ro1"]["b"].reshape(1, 1),
            params["atomic_e"]]
    tot = _tc_readout(nf, an2, owts)
    return tot[0, 0] * params["scale"] + params["shift"]


# speedup vs baseline: 5.1038x; 1.0332x over previous
import functools
import math

import jax
import jax.numpy as jnp
from jax import lax
from jax.experimental import pallas as pl
from jax.experimental.pallas import tpu as pltpu
from jax.experimental.pallas import tpu_sc as plsc

N = 50000
E = 800000
H = 64
HH = H // 2
NB = 8
NUM_ATOMS = 100
CUTOFF = 5.0

NC = 2
NS = 16
NW = NC * NS
CH = 128
NCHUNKS = E // CH
ROWS_A = 3128
ROWS_LAST = N - (NS - 1) * ROWS_A

BN = 2000
BR = 64
EP = 819200
ER = EP // 128

_SC_MESH = plsc.VectorSubcoreMesh(
    core_axis_name="c", subcore_axis_name="s", num_cores=NC, num_subcores=NS
)


def _silu(x):
    return x * (1.0 / (1.0 + jnp.exp(-x)))


_SC_PARAMS = pltpu.CompilerParams(use_tc_tiling_on_sc=False)
_D2_SLOTS = NCHUNKS // NW + 3


def _sc_edge_d2_body(ei_h, px_h, py_h, pz_h, d2_h, *sc):
    c = lax.axis_index("c")
    s = lax.axis_index("s")
    wid = s * NC + c
    names = ("rowi", "coli", "g", "d2v", "sidx", "sg", "swr")
    A = dict(zip(names, sc[0:7]))
    B = dict(zip(names, sc[7:14]))

    def valid(j):
        return jnp.logical_and(j >= 0, wid + j * NW < NCHUNKS)

    def chunk_of(j):
        cid = wid + j * NW
        return jnp.where(valid(j), cid, 0)

    def base_of(j):
        return pl.multiple_of(chunk_of(j) * CH, CH)

    def issue_idx(j, X):
        bs = base_of(j)
        pltpu.async_copy(ei_h.at[0, pl.ds(bs, CH)], X["rowi"], X["sidx"])
        pltpu.async_copy(ei_h.at[1, pl.ds(bs, CH)], X["coli"], X["sidx"])

    def wait_idx(X):
        pltpu.make_async_copy(ei_h.at[0, pl.ds(0, CH)], X["rowi"], X["sidx"]).wait()
        pltpu.make_async_copy(ei_h.at[1, pl.ds(0, CH)], X["coli"], X["sidx"]).wait()

    def issue_gathers(X):
        for k, (tb, ib) in enumerate(((px_h, "rowi"), (py_h, "rowi"),
                                      (pz_h, "rowi"), (px_h, "coli"),
                                      (py_h, "coli"), (pz_h, "coli"))):
            pltpu.async_copy(tb.at[X[ib]], X["g"].at[k], X["sg"])

    def wait_gathers(X):
        for k in range(6):
            pltpu.make_async_copy(px_h.at[X["rowi"]], X["g"].at[k],
                                  X["sg"]).wait()

    def compute_and_write(j, X):
        g = X["g"]
        d2v = X["d2v"]
        for t in range(CH // 16):
            sl = pl.ds(t * 16, 16)
            dx = g[0, sl] - g[3, sl]
            dy = g[1, sl] - g[4, sl]
            dz = g[2, sl] - g[5, sl]
            d2v[sl] = dx * dx + dy * dy + dz * dz
        pltpu.async_copy(d2v, d2_h.at[chunk_of(j)], X["swr"])

    def wait_write(X):
        pltpu.make_async_copy(X["d2v"], d2_h.at[0], X["swr"]).wait()

    def half(i, CUR, NXT):
        @pl.when(valid(i))
        def _():
            wait_idx(CUR)

        @pl.when(valid(i - 2))
        def _():
            wait_write(CUR)

        @pl.when(valid(i))
        def _():
            issue_gathers(CUR)

        @pl.when(valid(i - 1))
        def _():
            wait_gathers(NXT)
            compute_and_write(i - 1, NXT)

        @pl.when(valid(i + 1))
        def _():
            issue_idx(i + 1, NXT)

    issue_idx(0, A)

    @pl.loop(0, _D2_SLOTS // 2)
    def _pair(k):
        half(2 * k, A, B)
        half(2 * k + 1, B, A)

    @pl.when(valid(_D2_SLOTS - 2))
    def _():
        wait_write(A)

    @pl.when(valid(_D2_SLOTS - 1))
    def _():
        wait_write(B)


def _d2_scratch_set():
    return [
        pltpu.VMEM((CH,), jnp.int32),
        pltpu.VMEM((CH,), jnp.int32),
        pltpu.VMEM((6, CH), jnp.float32),
        pltpu.VMEM((CH,), jnp.float32),
        pltpu.SemaphoreType.DMA,
        pltpu.SemaphoreType.DMA,
        pltpu.SemaphoreType.DMA,
    ]


_sc_edge_d2 = functools.partial(
    pl.kernel,
    out_type=jax.ShapeDtypeStruct((ER, 128), jnp.float32),
    mesh=_SC_MESH,
    compiler_params=_SC_PARAMS,
    scratch_types=_d2_scratch_set() + _d2_scratch_set(),
)(_sc_edge_d2_body)


_MSG_SLOTS = NCHUNKS // NS + 4


def _sc_messages_body(ei_h, nf_h, rw_h, zer_h, agg_h, *sc):
    c = lax.axis_index("c")
    s = lax.axis_index("s")
    names = ("rowi", "rowsc", "coli", "nj", "rwv", "sidx", "sg", "sw", "ssc")
    A = dict(zip(names, sc[0:9]))
    B = dict(zip(names, sc[9:18]))
    agg_sh = sc[18]
    nf_c = nf_h.at[c]
    rw_c = rw_h.at[c]
    agg_c = agg_h.at[c]

    rbase = pl.multiple_of(s * ROWS_A, 8)

    @pl.when(s < NS - 1)
    def _():
        pltpu.sync_copy(zer_h, agg_sh.at[pl.ds(rbase, ROWS_A)])

    @pl.when(s == NS - 1)
    def _():
        pltpu.sync_copy(zer_h.at[pl.ds(0, ROWS_LAST)],
                        agg_sh.at[pl.ds(rbase, ROWS_LAST)])

    plsc.subcore_barrier()

    def valid(j):
        return jnp.logical_and(j >= 0, s + j * NS < NCHUNKS)

    def base_of(j):
        cid = s + j * NS
        cid = jnp.where(valid(j), cid, 0)
        return pl.multiple_of(cid * CH, CH)

    def issue_idx(j, X):
        bs = base_of(j)
        pltpu.async_copy(ei_h.at[0, pl.ds(bs, CH)], X["rowi"], X["sidx"])
        pltpu.async_copy(ei_h.at[1, pl.ds(bs, CH)], X["coli"], X["sidx"])

    def wait_idx(X):
        pltpu.make_async_copy(ei_h.at[0, pl.ds(0, CH)], X["rowi"], X["sidx"]).wait()
        pltpu.make_async_copy(ei_h.at[1, pl.ds(0, CH)], X["coli"], X["sidx"]).wait()

    def issue_loads(j, X):
        pltpu.async_copy(nf_c.at[X["coli"]], X["nj"], X["sg"])
        pltpu.async_copy(rw_c.at[pl.ds(base_of(j), CH)], X["rwv"], X["sw"])

    def wait_loads(X):
        pltpu.make_async_copy(nf_c.at[X["coli"]], X["nj"], X["sg"]).wait()
        pltpu.make_async_copy(rw_c.at[pl.ds(0, CH)], X["rwv"], X["sw"]).wait()

    def mult_scatter(X):
        nj = X["nj"]
        rwv = X["rwv"]
        lo = pl.ds(0, 16)
        hi = pl.ds(16, 16)

        @pl.loop(0, CH, unroll=8)
        def _mul(j):
            nj[j, lo] = nj[j, lo] * rwv[j, lo]
            nj[j, hi] = nj[j, hi] * rwv[j, hi]

        for t in range(CH // 16):
            sl = pl.ds(t * 16, 16)
            X["rowsc"][sl] = X["rowi"][sl]
        pltpu.async_copy(nj, agg_sh.at[X["rowsc"]], X["ssc"], add=True)

    def wait_scatter(X):
        pltpu.make_async_copy(X["nj"], agg_sh.at[X["rowsc"]], X["ssc"]).wait()

    def half(i, CUR, NXT):
        @pl.when(valid(i))
        def _():
            wait_idx(CUR)

        @pl.when(valid(i - 2))
        def _():
            wait_scatter(CUR)

        @pl.when(valid(i))
        def _():
            issue_loads(i, CUR)

        @pl.when(valid(i - 1))
        def _():
            wait_loads(NXT)
            mult_scatter(NXT)

        @pl.when(valid(i + 1))
        def _():
            issue_idx(i + 1, NXT)

    issue_idx(0, A)

    @pl.loop(0, _MSG_SLOTS // 2)
    def _pair(k):
        half(2 * k, A, B)
        half(2 * k + 1, B, A)

    plsc.subcore_barrier()

    @pl.when(s < NS - 1)
    def _():
        pltpu.sync_copy(agg_sh.at[pl.ds(rbase, ROWS_A)],
                        agg_c.at[pl.ds(rbase, ROWS_A)])

    @pl.when(s == NS - 1)
    def _():
        pltpu.sync_copy(agg_sh.at[pl.ds(rbase, ROWS_LAST)],
                        agg_c.at[pl.ds(rbase, ROWS_LAST)])


def _msg_scratch_set():
    return [
        pltpu.VMEM((CH,), jnp.int32),
        pltpu.VMEM((CH,), jnp.int32),
        pltpu.VMEM((CH,), jnp.int32),
        pltpu.VMEM((CH, HH), jnp.float32),
        pltpu.VMEM((CH, HH), jnp.float32),
        pltpu.SemaphoreType.DMA,
        pltpu.SemaphoreType.DMA,
        pltpu.SemaphoreType.DMA,
        pltpu.SemaphoreType.DMA,
    ]


_sc_messages = functools.partial(
    pl.kernel,
    out_type=jax.ShapeDtypeStruct((2, N, HH), jnp.float32),
    mesh=_SC_MESH,
    compiler_params=_SC_PARAMS,
    scratch_types=_msg_scratch_set() + _msg_scratch_set()
                  + [pltpu.VMEM_SHARED((N, HH), jnp.float32)],
)(_sc_messages_body)


def _tc_emb_body(an_ref, emb_ref, nf_ref):
    an = an_ref[...]
    ids = lax.broadcasted_iota(jnp.int32, (BN, NUM_ATOMS), 1)
    oh = (an == ids).astype(jnp.float32)
    nf = jnp.dot(oh, emb_ref[...], preferred_element_type=jnp.float32)
    nf_ref[0, :, :] = nf[:, :HH]
    nf_ref[1, :, :] = nf[:, HH:]


def _tc_radial_body(d2_ref, w0, b0, w1, b1, w2, b2, o_ref):
    d2 = d2_ref[...]
    d = jnp.sqrt(d2)
    th = d * (math.pi / CUTOFF)
    s1 = jnp.sin(th)
    c1 = jnp.cos(th)
    cut = 0.5 * (c1 + 1.0)
    cut = cut * (d < CUTOFF).astype(jnp.float32)
    g = cut / jnp.clip(d, 1e-8, None)
    two_c = 2.0 * c1
    bs = []
    sk_m1 = jnp.zeros_like(s1)
    sk = s1
    for _ in range(NB):
        bs.append(sk * g)
        sk, sk_m1 = two_c * sk - sk_m1, sk
    rbf_t = jnp.stack(bs, axis=0).reshape(NB, BR * 128)
    rbf = jnp.transpose(rbf_t, (1, 0))
    h = _silu(jnp.dot(rbf, w0[...], preferred_element_type=jnp.float32)
              + b0[...])
    h = _silu(jnp.dot(h, w1[...], preferred_element_type=jnp.float32)
              + b1[...])
    rw = jnp.dot(h, w2[...], preferred_element_type=jnp.float32) + b2[...]
    o_ref[0, :, :] = rw[:, :HH]
    o_ref[1, :, :] = rw[:, HH:]


def _tc_update_body(nf_ref, agg_ref, wn_ref, wa_ref, b_ref, lnw_ref, lnb_ref,
                    out_ref):
    nf = jnp.concatenate([nf_ref[0], nf_ref[1]], axis=-1)
    ag = jnp.concatenate([agg_ref[0], agg_ref[1]], axis=-1)
    upd = (jnp.dot(nf, wn_ref[...], preferred_element_type=jnp.float32)
           + jnp.dot(ag, wa_ref[...], preferred_element_type=jnp.float32)
           + b_ref[...])
    x = nf + upd
    m = jnp.mean(x, axis=-1, keepdims=True)
    v = jnp.mean((x - m) ** 2, axis=-1, keepdims=True)
    y = (x - m) / jnp.sqrt(v + 1e-5) * lnw_ref[...] + lnb_ref[...]
    out_ref[0, :, :] = y[:, :HH]
    out_ref[1, :, :] = y[:, HH:]


def _tc_readout_body(nf_ref, an_ref, w0_ref, b0_ref, w1_ref, b1_ref, ae_ref,
                     out_ref):
    nf = jnp.concatenate([nf_ref[0], nf_ref[1]], axis=-1)
    t = _silu(jnp.dot(nf, w0_ref[...], preferred_element_type=jnp.float32)
              + b0_ref[...])
    e = jnp.dot(t, w1_ref[...], preferred_element_type=jnp.float32) + b1_ref[...]
    an = an_ref[...]
    ids = lax.broadcasted_iota(jnp.int32, (BN, NUM_ATOMS), 1)
    oh = (an == ids).astype(jnp.float32)
    e = e + jnp.dot(oh, ae_ref[...], preferred_element_type=jnp.float32)

    @pl.when(pl.program_id(0) == 0)
    def _():
        out_ref[...] = jnp.zeros_like(out_ref)

    out_ref[...] = out_ref[...] + jnp.sum(e).reshape(1, 1)


def _full(shape):
    return pl.BlockSpec(shape, lambda i: tuple(0 for _ in shape))


def _tc_emb(an2, emb):
    return pl.pallas_call(
        _tc_emb_body,
        grid=(N // BN,),
        in_specs=[pl.BlockSpec((BN, 1), lambda i: (i, 0)),
                  _full((NUM_ATOMS, H))],
        out_specs=pl.BlockSpec((2, BN, HH), lambda i: (0, i, 0)),
        out_shape=jax.ShapeDtypeStruct((2, N, HH), jnp.float32),
    )(an2, emb)


def _tc_radial(d2, wts):
    return pl.pallas_call(
        _tc_radial_body,
        grid=(ER // BR,),
        in_specs=[pl.BlockSpec((BR, 128), lambda i: (i, 0))]
                 + [_full(w.shape) for w in wts],
        out_specs=pl.BlockSpec((2, BR * 128, HH), lambda i: (0, i, 0)),
        out_shape=jax.ShapeDtypeStruct((2, EP, HH), jnp.float32),
    )(d2, *wts)


def _tc_update(nf, agg, wts):
    return pl.pallas_call(
        _tc_update_body,
        grid=(N // BN,),
        in_specs=[pl.BlockSpec((2, BN, HH), lambda i: (0, i, 0))] * 2
                 + [_full(w.shape) for w in wts],
        out_specs=pl.BlockSpec((2, BN, HH), lambda i: (0, i, 0)),
        out_shape=jax.ShapeDtypeStruct((2, N, HH), jnp.float32),
    )(nf, agg, *wts)


def _tc_readout(nf, an2, wts):
    return pl.pallas_call(
        _tc_readout_body,
        grid=(N // BN,),
        in_specs=[pl.BlockSpec((2, BN, HH), lambda i: (0, i, 0)),
                  pl.BlockSpec((BN, 1), lambda i: (i, 0))]
                 + [_full(w.shape) for w in wts],
        out_specs=pl.BlockSpec((1, 1), lambda i: (0, 0)),
        out_shape=jax.ShapeDtypeStruct((1, 1), jnp.float32),
    )(nf, an2, *wts)


def kernel(atomic_numbers, pos, edge_index, params):
    px, py, pz = pos[:, 0], pos[:, 1], pos[:, 2]
    an2 = atomic_numbers.reshape(N, 1)

    d2 = _sc_edge_d2(edge_index, px, py, pz)
    nf = _tc_emb(an2, params["emb"])

    zer = jnp.zeros((ROWS_A, HH), jnp.float32)
    for li, lp in enumerate(params["layers"]):
        w2f = lp["rn2"]["w"].reshape(H, H, 3).sum(-1)
        b2f = lp["rn2"]["b"].reshape(H, 3).sum(-1)
        rw = _tc_radial(d2, [lp["rn0"]["w"], lp["rn0"]["b"].reshape(1, H),
                             lp["rn1"]["w"], lp["rn1"]["b"].reshape(1, H),
                             w2f, b2f.reshape(1, H)])
        agg = _sc_messages(edge_index, nf, rw, zer)
        uwts = [lp["lin"]["w"][:H], lp["lin"]["w"][H:],
                lp["lin"]["b"].reshape(1, H),
                lp["ln_w"].reshape(1, H), lp["ln_b"].reshape(1, H)]
        nf = _tc_update(nf, agg, uwts)

    owts = [params["ro0"]["w"], params["ro0"]["b"].reshape(1, H),
            params["ro1"]["w"], params["ro1"]["b"].reshape(1, 1),
            params["atomic_e"]]
    tot = _tc_readout(nf, an2, owts)
    return tot[0, 0] * params["scale"] + params["shift"]
